# baseline (speedup 1.0000x reference)
import functools

import jax
import jax.numpy as jnp
from jax import lax
from jax.experimental import pallas as pl
from jax.experimental.pallas import tpu as pltpu
from jax.experimental.pallas import tpu_sc as plsc

N = 50000
E = 800000
NODE_IN = 163
H = 64
G = 64

RB = 400
NB = N // RB

CH = 128
NCH = E // CH
N_SC = 2
N_TILE = 16
IT_DEG = -(-NCH // (N_SC * N_TILE))

EB = 4000
NEB = E // EB
NGB = EB // 16

_f32 = jnp.float32


def _leaky(v):
    return jnp.where(v >= 0, v, 0.01 * v)


_MESH = plsc.VectorSubcoreMesh(core_axis_name="c", subcore_axis_name="s")


@functools.partial(
    pl.kernel,
    out_type=[
        jax.ShapeDtypeStruct((N,), _f32),
        jax.ShapeDtypeStruct((N,), _f32),
    ],
    mesh=_MESH,
    scratch_types=[
        pltpu.VMEM((CH,), jnp.int32),
        pltpu.VMEM((CH,), _f32),
        pltpu.VMEM_SHARED((N,), _f32),
    ],
)
def _deg_kernel(dst1, ones_h, zeros_h, deg_a, deg_b, dstbuf, onesbuf, deg_sh):
    c = lax.axis_index("c")
    s = lax.axis_index("s")
    @pl.when(s == 0)
    def _():
        pltpu.sync_copy(zeros_h, deg_sh)
    pltpu.sync_copy(ones_h, onesbuf)
    plsc.subcore_barrier()
    w = c * N_TILE + s

    def body(k, carry):
        cid = w + k * (N_SC * N_TILE)
        @pl.when(cid < NCH)
        def _():
            pltpu.sync_copy(dst1.at[pl.ds(cid * CH, CH)], dstbuf)
            pltpu.sync_copy(onesbuf, deg_sh.at[dstbuf], add=True)
        return carry

    lax.fori_loop(0, IT_DEG, body, 0)
    plsc.subcore_barrier()
    @pl.when(jnp.logical_and(s == 0, c == 0))
    def _():
        pltpu.sync_copy(deg_sh, deg_a)
    @pl.when(jnp.logical_and(s == 0, c == 1))
    def _():
        pltpu.sync_copy(deg_sh, deg_b)


@functools.partial(
    pl.kernel,
    out_type=jax.ShapeDtypeStruct((H * N,), _f32),
    mesh=_MESH,
    compiler_params=pltpu.CompilerParams(needs_layout_passes=False),
    scratch_types=[
        pltpu.VMEM((N,), _f32),
        pltpu.VMEM((N,), _f32),
        pltpu.VMEM((EB,), jnp.int32),
        pltpu.VMEM((EB,), jnp.int32),
        pltpu.SemaphoreType.DMA,
        pltpu.SemaphoreType.DMA,
    ],
)
def _prop_kernel(xs_cm, pse, acc_cm, table, acc, eb0, eb1, sem0, sem1):
    c = lax.axis_index("c")
    s = lax.axis_index("s")
    w = c * N_TILE + s
    bufs = ((eb0, sem0), (eb1, sem1))

    def fetch(k, b):
        pb, sem = bufs[b]
        pltpu.async_copy(pse.at[pl.ds(k * EB, EB)], pb, sem)

    def drain(b):
        pb, sem = bufs[b]
        pltpu.make_async_copy(pse.at[pl.ds(0, EB)], pb, sem).wait()

    def process(b):
        pb, _ = bufs[b]

        def grp(g, carry2):
            for u in range(5):
                p = pb[pl.ds((g * 5 + u) * 16, 16)]
                vs = lax.shift_right_logical(p, 16)
                vd = jnp.bitwise_and(p, 0xFFFF)
                vals = plsc.load_gather(table, [vs])
                plsc.addupdate_scatter(acc, [vd], vals)
            return carry2

        lax.fori_loop(0, NGB // 5, grp, 0)

    for r in range(2):
        fsl = pl.ds((w + 32 * r) * N, N)
        pltpu.sync_copy(xs_cm.at[fsl], table)
        pltpu.sync_copy(xs_cm.at[fsl], acc)

        fetch(0, 0)

        def chunk2(k2, carry):
            fetch(2 * k2 + 1, 1)
            drain(0)
            process(0)
            @pl.when(k2 + 1 < NEB // 2)
            def _():
                fetch(2 * k2 + 2, 0)
            drain(1)
            process(1)
            return carry

        lax.fori_loop(0, NEB // 2, chunk2, 0)
        pltpu.sync_copy(acc, acc_cm.at[fsl])


def _mm1_body(x_ref, w_ref, dega_ref, degb_ref, xs_ref, dinv_ref):
    deg = dega_ref[0, 0, :] + degb_ref[0, 0, :] + 1.0
    dinv = lax.rsqrt(deg)
    xw = jnp.dot(x_ref[...], w_ref[...], preferred_element_type=_f32)
    xs_ref[...] = xw * dinv[:, None]
    dinv_ref[0, 0, :] = dinv


def _mm1(x, w1, dega3, degb3):
    return pl.pallas_call(
        _mm1_body,
        grid=(NB,),
        in_specs=[
            pl.BlockSpec((RB, NODE_IN), lambda i: (i, 0)),
            pl.BlockSpec((NODE_IN, H), lambda i: (0, 0)),
            pl.BlockSpec((1, 1, RB), lambda i: (i, 0, 0)),
            pl.BlockSpec((1, 1, RB), lambda i: (i, 0, 0)),
        ],
        out_specs=[
            pl.BlockSpec((RB, H), lambda i: (i, 0)),
            pl.BlockSpec((1, 1, RB), lambda i: (i, 0, 0)),
        ],
        out_shape=[
            jax.ShapeDtypeStruct((N, H), _f32),
            jax.ShapeDtypeStruct((NB, 1, RB), _f32),
        ],
    )(x, w1, dega3, degb3)


def _mm23_body(acc_ref, dinv_ref, b_ref, w_ref, xs_ref):
    dinv = dinv_ref[0, 0, :]
    h = _leaky(acc_ref[...] * dinv[:, None] + b_ref[0, :])
    xw = jnp.dot(h, w_ref[...], preferred_element_type=_f32)
    xs_ref[...] = xw * dinv[:, None]


def _mm23(acc, dinv3, b, w):
    return pl.pallas_call(
        _mm23_body,
        grid=(NB,),
        in_specs=[
            pl.BlockSpec((RB, H), lambda i: (i, 0)),
            pl.BlockSpec((1, 1, RB), lambda i: (i, 0, 0)),
            pl.BlockSpec((1, H), lambda i: (0, 0)),
            pl.BlockSpec((H, H), lambda i: (0, 0)),
        ],
        out_specs=pl.BlockSpec((RB, H), lambda i: (i, 0)),
        out_shape=jax.ShapeDtypeStruct((N, H), _f32),
    )(acc, dinv3, b, w)


def _pool_body(acc_ref, dinv_ref, b3_ref, batch_ref,
               wf1_ref, bf1_ref, wf2t_ref, bf2_ref, out_ref, sums, cnt):
    i = pl.program_id(0)

    @pl.when(i == 0)
    def _():
        sums[...] = jnp.zeros_like(sums)
        cnt[...] = jnp.zeros_like(cnt)

    dinv = dinv_ref[0, 0, :]
    h = _leaky(acc_ref[...] * dinv[:, None] + b3_ref[0, :])
    seg = batch_ref[0, 0, :]
    onehot = (seg[:, None]
              == lax.broadcasted_iota(jnp.int32, (RB, G), 1)).astype(_f32)
    sums[...] += lax.dot_general(onehot, h, (((0,), (0,)), ((), ())),
                                 preferred_element_type=_f32)
    cnt[...] += jnp.sum(onehot, axis=0, keepdims=True)

    @pl.when(i == NB - 1)
    def _():
        pooled = sums[...] / jnp.maximum(cnt[0, :], 1.0)[:, None]
        hh = _leaky(jnp.dot(pooled, wf1_ref[...],
                            preferred_element_type=_f32) + bf1_ref[0, :])
        o = jnp.sum(hh * wf2t_ref[0, :][None, :], axis=1) + bf2_ref[0, 0]
        out_ref[0, :] = o


def _pool(acc, dinv3, b3, batch3, wf1, bf1, wf2t, bf2):
    return pl.pallas_call(
        _pool_body,
        grid=(NB,),
        in_specs=[
            pl.BlockSpec((RB, H), lambda i: (i, 0)),
            pl.BlockSpec((1, 1, RB), lambda i: (i, 0, 0)),
            pl.BlockSpec((1, H), lambda i: (0, 0)),
            pl.BlockSpec((1, 1, RB), lambda i: (i, 0, 0)),
            pl.BlockSpec((H, H), lambda i: (0, 0)),
            pl.BlockSpec((1, H), lambda i: (0, 0)),
            pl.BlockSpec((1, H), lambda i: (0, 0)),
            pl.BlockSpec((1, 1), lambda i: (0, 0)),
        ],
        out_specs=pl.BlockSpec((1, G), lambda i: (0, 0)),
        out_shape=jax.ShapeDtypeStruct((1, G), _f32),
        scratch_shapes=[
            pltpu.VMEM((G, H), _f32),
            pltpu.VMEM((1, G), _f32),
        ],
    )(acc, dinv3, b3, batch3, wf1, bf1, wf2t, bf2)


def kernel(x, edge_index, batch, W1, b1, W2, b2, W3, b3, Wf1, bf1, Wf2, bf2):
    src1 = edge_index[0]
    dst1 = edge_index[1]
    ones_h = jnp.ones((CH,), _f32)
    zeros_h = jnp.zeros((N,), _f32)

    deg_a, deg_b = _deg_kernel(dst1, ones_h, zeros_h)
    xs, dinv3 = _mm1(x, W1, deg_a.reshape(NB, 1, RB), deg_b.reshape(NB, 1, RB))

    packed = jnp.bitwise_or(jnp.left_shift(src1, 16), dst1)

    def prop(xs):
        acc_cm = _prop_kernel(xs.T.reshape(H * N), packed)
        return acc_cm.reshape(H, N).T

    acc = prop(xs)
    xs = _mm23(acc, dinv3, b1.reshape(1, H), W2)
    acc = prop(xs)
    xs = _mm23(acc, dinv3, b2.reshape(1, H), W3)
    acc = prop(xs)

    out = _pool(acc, dinv3, b3.reshape(1, H),
                batch.reshape(NB, 1, RB), Wf1, bf1.reshape(1, H),
                Wf2.reshape(1, H), bf2.reshape(1, 1))
    return out.reshape(G)


# speedup vs baseline: 1.1058x; 1.1058x over previous
import functools

import jax
import jax.numpy as jnp
from jax import lax
from jax.experimental import pallas as pl
from jax.experimental.pallas import tpu as pltpu
from jax.experimental.pallas import tpu_sc as plsc

N = 50000
E = 800000
NODE_IN = 163
H = 64
G = 64

RB = 400
NB = N // RB

CH = 128
NCH = E // CH
N_SC = 2
N_TILE = 16
IT_DEG = -(-NCH // (N_SC * N_TILE))

EB = 4000
NEB = E // EB
NGB = EB // 16

_f32 = jnp.float32


def _leaky(v):
    return jnp.where(v >= 0, v, 0.01 * v)


_MESH = plsc.VectorSubcoreMesh(core_axis_name="c", subcore_axis_name="s")


@functools.partial(
    pl.kernel,
    out_type=[
        jax.ShapeDtypeStruct((N,), _f32),
        jax.ShapeDtypeStruct((N,), _f32),
    ],
    mesh=_MESH,
    scratch_types=[
        pltpu.VMEM((CH,), jnp.int32),
        pltpu.VMEM((CH,), _f32),
        pltpu.VMEM_SHARED((N,), _f32),
    ],
)
def _deg_kernel(dst1, ones_h, zeros_h, deg_a, deg_b, dstbuf, onesbuf, deg_sh):
    c = lax.axis_index("c")
    s = lax.axis_index("s")
    @pl.when(s == 0)
    def _():
        pltpu.sync_copy(zeros_h, deg_sh)
    pltpu.sync_copy(ones_h, onesbuf)
    plsc.subcore_barrier()
    w = c * N_TILE + s

    def body(k, carry):
        cid = w + k * (N_SC * N_TILE)
        @pl.when(cid < NCH)
        def _():
            pltpu.sync_copy(dst1.at[pl.ds(cid * CH, CH)], dstbuf)
            pltpu.sync_copy(onesbuf, deg_sh.at[dstbuf], add=True)
        return carry

    lax.fori_loop(0, IT_DEG, body, 0)
    plsc.subcore_barrier()
    @pl.when(jnp.logical_and(s == 0, c == 0))
    def _():
        pltpu.sync_copy(deg_sh, deg_a)
    @pl.when(jnp.logical_and(s == 0, c == 1))
    def _():
        pltpu.sync_copy(deg_sh, deg_b)


@functools.partial(
    pl.kernel,
    out_type=jax.ShapeDtypeStruct((H * N,), _f32),
    mesh=_MESH,
    compiler_params=pltpu.CompilerParams(needs_layout_passes=False),
    scratch_types=[
        pltpu.VMEM((N,), _f32),
        pltpu.VMEM((N,), _f32),
        pltpu.VMEM((EB,), jnp.int32),
        pltpu.VMEM((EB,), jnp.int32),
        pltpu.VMEM((EB,), jnp.int32),
        pltpu.VMEM((EB,), jnp.int32),
        pltpu.SemaphoreType.DMA,
        pltpu.SemaphoreType.DMA,
    ],
)
def _prop_kernel(xs_cm, src1, dst1, acc_cm, table, acc,
                 srcb0, dstb0, srcb1, dstb1, sem0, sem1):
    c = lax.axis_index("c")
    s = lax.axis_index("s")
    w = c * N_TILE + s
    bufs = ((srcb0, dstb0, sem0), (srcb1, dstb1, sem1))

    def fetch(k, b):
        sb, db, sem = bufs[b]
        eb = pl.ds(k * EB, EB)
        pltpu.async_copy(src1.at[eb], sb, sem)
        pltpu.async_copy(dst1.at[eb], db, sem)

    def drain(b):
        sb, db, sem = bufs[b]
        pltpu.make_async_copy(src1.at[pl.ds(0, EB)], sb, sem).wait()
        pltpu.make_async_copy(dst1.at[pl.ds(0, EB)], db, sem).wait()

    def process(b):
        sb, db, _ = bufs[b]

        def grp(g, carry2):
            for u in range(10):
                gs = pl.ds((g * 10 + u) * 16, 16)
                vals = plsc.load_gather(table, [sb[gs]])
                plsc.addupdate_scatter(acc, [db[gs]], vals)
            return carry2

        lax.fori_loop(0, NGB // 10, grp, 0)

    for r in range(2):
        fsl = pl.ds((w + 32 * r) * N, N)
        pltpu.sync_copy(xs_cm.at[fsl], table)
        pltpu.sync_copy(xs_cm.at[fsl], acc)

        fetch(0, 0)

        def chunk2(k2, carry):
            fetch(2 * k2 + 1, 1)
            drain(0)
            process(0)
            @pl.when(k2 + 1 < NEB // 2)
            def _():
                fetch(2 * k2 + 2, 0)
            drain(1)
            process(1)
            return carry

        lax.fori_loop(0, NEB // 2, chunk2, 0)
        pltpu.sync_copy(acc, acc_cm.at[fsl])


def _mm1_body(x_ref, w_ref, dega_ref, degb_ref, xs_ref, dinv_ref):
    deg = dega_ref[0, 0, :] + degb_ref[0, 0, :] + 1.0
    dinv = lax.rsqrt(deg)
    xw = jnp.dot(x_ref[...], w_ref[...], preferred_element_type=_f32)
    xs_ref[...] = xw * dinv[:, None]
    dinv_ref[0, 0, :] = dinv


def _mm1(x, w1, dega3, degb3):
    return pl.pallas_call(
        _mm1_body,
        grid=(NB,),
        in_specs=[
            pl.BlockSpec((RB, NODE_IN), lambda i: (i, 0)),
            pl.BlockSpec((NODE_IN, H), lambda i: (0, 0)),
            pl.BlockSpec((1, 1, RB), lambda i: (i, 0, 0)),
            pl.BlockSpec((1, 1, RB), lambda i: (i, 0, 0)),
        ],
        out_specs=[
            pl.BlockSpec((RB, H), lambda i: (i, 0)),
            pl.BlockSpec((1, 1, RB), lambda i: (i, 0, 0)),
        ],
        out_shape=[
            jax.ShapeDtypeStruct((N, H), _f32),
            jax.ShapeDtypeStruct((NB, 1, RB), _f32),
        ],
    )(x, w1, dega3, degb3)


def _mm23_body(acc_ref, dinv_ref, b_ref, w_ref, xs_ref):
    dinv = dinv_ref[0, 0, :]
    h = _leaky(acc_ref[...] * dinv[:, None] + b_ref[0, :])
    xw = jnp.dot(h, w_ref[...], preferred_element_type=_f32)
    xs_ref[...] = xw * dinv[:, None]


def _mm23(acc, dinv3, b, w):
    return pl.pallas_call(
        _mm23_body,
        grid=(NB,),
        in_specs=[
            pl.BlockSpec((RB, H), lambda i: (i, 0)),
            pl.BlockSpec((1, 1, RB), lambda i: (i, 0, 0)),
            pl.BlockSpec((1, H), lambda i: (0, 0)),
            pl.BlockSpec((H, H), lambda i: (0, 0)),
        ],
        out_specs=pl.BlockSpec((RB, H), lambda i: (i, 0)),
        out_shape=jax.ShapeDtypeStruct((N, H), _f32),
    )(acc, dinv3, b, w)


def _pool_body(acc_ref, dinv_ref, b3_ref, batch_ref,
               wf1_ref, bf1_ref, wf2t_ref, bf2_ref, out_ref, sums, cnt):
    i = pl.program_id(0)

    @pl.when(i == 0)
    def _():
        sums[...] = jnp.zeros_like(sums)
        cnt[...] = jnp.zeros_like(cnt)

    dinv = dinv_ref[0, 0, :]
    h = _leaky(acc_ref[...] * dinv[:, None] + b3_ref[0, :])
    seg = batch_ref[0, 0, :]
    onehot = (seg[:, None]
              == lax.broadcasted_iota(jnp.int32, (RB, G), 1)).astype(_f32)
    sums[...] += lax.dot_general(onehot, h, (((0,), (0,)), ((), ())),
                                 preferred_element_type=_f32)
    cnt[...] += jnp.sum(onehot, axis=0, keepdims=True)

    @pl.when(i == NB - 1)
    def _():
        pooled = sums[...] / jnp.maximum(cnt[0, :], 1.0)[:, None]
        hh = _leaky(jnp.dot(pooled, wf1_ref[...],
                            preferred_element_type=_f32) + bf1_ref[0, :])
        o = jnp.sum(hh * wf2t_ref[0, :][None, :], axis=1) + bf2_ref[0, 0]
        out_ref[0, :] = o


def _pool(acc, dinv3, b3, batch3, wf1, bf1, wf2t, bf2):
    return pl.pallas_call(
        _pool_body,
        grid=(NB,),
        in_specs=[
            pl.BlockSpec((RB, H), lambda i: (i, 0)),
            pl.BlockSpec((1, 1, RB), lambda i: (i, 0, 0)),
            pl.BlockSpec((1, H), lambda i: (0, 0)),
            pl.BlockSpec((1, 1, RB), lambda i: (i, 0, 0)),
            pl.BlockSpec((H, H), lambda i: (0, 0)),
            pl.BlockSpec((1, H), lambda i: (0, 0)),
            pl.BlockSpec((1, H), lambda i: (0, 0)),
            pl.BlockSpec((1, 1), lambda i: (0, 0)),
        ],
        out_specs=pl.BlockSpec((1, G), lambda i: (0, 0)),
        out_shape=jax.ShapeDtypeStruct((1, G), _f32),
        scratch_shapes=[
            pltpu.VMEM((G, H), _f32),
            pltpu.VMEM((1, G), _f32),
        ],
    )(acc, dinv3, b3, batch3, wf1, bf1, wf2t, bf2)


def kernel(x, edge_index, batch, W1, b1, W2, b2, W3, b3, Wf1, bf1, Wf2, bf2):
    src1 = edge_index[0]
    dst1 = edge_index[1]
    ones_h = jnp.ones((CH,), _f32)
    zeros_h = jnp.zeros((N,), _f32)

    deg_a, deg_b = _deg_kernel(dst1, ones_h, zeros_h)
    xs, dinv3 = _mm1(x, W1, deg_a.reshape(NB, 1, RB), deg_b.reshape(NB, 1, RB))

    def prop(xs):
        acc_cm = _prop_kernel(xs.T.reshape(H * N), src1, dst1)
        return acc_cm.reshape(H, N).T

    acc = prop(xs)
    xs = _mm23(acc, dinv3, b1.reshape(1, H), W2)
    acc = prop(xs)
    xs = _mm23(acc, dinv3, b2.reshape(1, H), W3)
    acc = prop(xs)

    out = _pool(acc, dinv3, b3.reshape(1, H),
                batch.reshape(NB, 1, RB), Wf1, bf1.reshape(1, H),
                Wf2.reshape(1, H), bf2.reshape(1, 1))
    return out.reshape(G)


# speedup vs baseline: 1.4188x; 1.2830x over previous
import functools

import jax
import jax.numpy as jnp
from jax import lax
from jax.experimental import pallas as pl
from jax.experimental.pallas import tpu as pltpu
from jax.experimental.pallas import tpu_sc as plsc

N = 50000
E = 800000
NODE_IN = 163
H = 64
G = 64

RB = 400
NB = N // RB

CH = 128
NCH = E // CH
N_SC = 2
N_TILE = 16
IT_DEG = -(-NCH // (N_SC * N_TILE))

EB = 4000
NEB = E // EB
NGB = EB // 16

_f32 = jnp.float32


def _leaky(v):
    return jnp.where(v >= 0, v, 0.01 * v)


_MESH = plsc.VectorSubcoreMesh(core_axis_name="c", subcore_axis_name="s")


@functools.partial(
    pl.kernel,
    out_type=[
        jax.ShapeDtypeStruct((N,), _f32),
        jax.ShapeDtypeStruct((N,), _f32),
    ],
    mesh=_MESH,
    scratch_types=[
        pltpu.VMEM((CH,), jnp.int32),
        pltpu.VMEM((CH,), _f32),
        pltpu.VMEM_SHARED((N,), _f32),
    ],
)
def _deg_kernel(dst1, ones_h, zeros_h, deg_a, deg_b, dstbuf, onesbuf, deg_sh):
    c = lax.axis_index("c")
    s = lax.axis_index("s")
    @pl.when(s == 0)
    def _():
        pltpu.sync_copy(zeros_h, deg_sh)
    pltpu.sync_copy(ones_h, onesbuf)
    plsc.subcore_barrier()
    w = c * N_TILE + s

    def body(k, carry):
        cid = w + k * (N_SC * N_TILE)
        @pl.when(cid < NCH)
        def _():
            pltpu.sync_copy(dst1.at[pl.ds(cid * CH, CH)], dstbuf)
            pltpu.sync_copy(onesbuf, deg_sh.at[dstbuf], add=True)
        return carry

    lax.fori_loop(0, IT_DEG, body, 0)
    plsc.subcore_barrier()
    @pl.when(jnp.logical_and(s == 0, c == 0))
    def _():
        pltpu.sync_copy(deg_sh, deg_a)
    @pl.when(jnp.logical_and(s == 0, c == 1))
    def _():
        pltpu.sync_copy(deg_sh, deg_b)


@functools.partial(
    pl.kernel,
    out_type=jax.ShapeDtypeStruct((H * N,), _f32),
    mesh=_MESH,
    compiler_params=pltpu.CompilerParams(needs_layout_passes=False),
    scratch_types=[
        pltpu.VMEM((N,), _f32),
        pltpu.VMEM((N,), _f32),
        pltpu.VMEM((EB,), jnp.int32),
        pltpu.VMEM((EB,), jnp.int32),
        pltpu.VMEM((EB,), jnp.int32),
        pltpu.VMEM((EB,), jnp.int32),
        pltpu.SemaphoreType.DMA,
        pltpu.SemaphoreType.DMA,
    ],
)
def _prop_kernel(xs_cm, src1, dst1, acc_cm, table, acc,
                 srcb0, dstb0, srcb1, dstb1, sem0, sem1):
    c = lax.axis_index("c")
    s = lax.axis_index("s")
    w = c * N_TILE + s
    bufs = ((srcb0, dstb0, sem0), (srcb1, dstb1, sem1))

    def fetch(k, b):
        sb, db, sem = bufs[b]
        eb = pl.ds(k * EB, EB)
        pltpu.async_copy(src1.at[eb], sb, sem)
        pltpu.async_copy(dst1.at[eb], db, sem)

    def drain(b):
        sb, db, sem = bufs[b]
        pltpu.make_async_copy(src1.at[pl.ds(0, EB)], sb, sem).wait()
        pltpu.make_async_copy(dst1.at[pl.ds(0, EB)], db, sem).wait()

    def process(b):
        sb, db, _ = bufs[b]

        def grp(g, carry2):
            vals = []
            for u in range(10):
                gs = pl.ds((g * 10 + u) * 16, 16)
                vals.append(plsc.load_gather(table, [sb[gs]]))
            for u in range(10):
                gs = pl.ds((g * 10 + u) * 16, 16)
                plsc.addupdate_scatter(acc, [db[gs]], vals[u])
            return carry2

        lax.fori_loop(0, NGB // 10, grp, 0)

    for r in range(2):
        fsl = pl.ds((w + 32 * r) * N, N)
        pltpu.sync_copy(xs_cm.at[fsl], table)
        pltpu.sync_copy(xs_cm.at[fsl], acc)

        fetch(0, 0)

        def chunk2(k2, carry):
            fetch(2 * k2 + 1, 1)
            drain(0)
            process(0)
            @pl.when(k2 + 1 < NEB // 2)
            def _():
                fetch(2 * k2 + 2, 0)
            drain(1)
            process(1)
            return carry

        lax.fori_loop(0, NEB // 2, chunk2, 0)
        pltpu.sync_copy(acc, acc_cm.at[fsl])


def _mm1_body(x_ref, w_ref, dega_ref, degb_ref, xs_ref, dinv_ref):
    deg = dega_ref[0, 0, :] + degb_ref[0, 0, :] + 1.0
    dinv = lax.rsqrt(deg)
    xw = jnp.dot(x_ref[...], w_ref[...], preferred_element_type=_f32)
    xs_ref[...] = xw * dinv[:, None]
    dinv_ref[0, 0, :] = dinv


def _mm1(x, w1, dega3, degb3):
    return pl.pallas_call(
        _mm1_body,
        grid=(NB,),
        in_specs=[
            pl.BlockSpec((RB, NODE_IN), lambda i: (i, 0)),
            pl.BlockSpec((NODE_IN, H), lambda i: (0, 0)),
            pl.BlockSpec((1, 1, RB), lambda i: (i, 0, 0)),
            pl.BlockSpec((1, 1, RB), lambda i: (i, 0, 0)),
        ],
        out_specs=[
            pl.BlockSpec((RB, H), lambda i: (i, 0)),
            pl.BlockSpec((1, 1, RB), lambda i: (i, 0, 0)),
        ],
        out_shape=[
            jax.ShapeDtypeStruct((N, H), _f32),
            jax.ShapeDtypeStruct((NB, 1, RB), _f32),
        ],
    )(x, w1, dega3, degb3)


def _mm23_body(acc_ref, dinv_ref, b_ref, w_ref, xs_ref):
    dinv = dinv_ref[0, 0, :]
    h = _leaky(acc_ref[...] * dinv[:, None] + b_ref[0, :])
    xw = jnp.dot(h, w_ref[...], preferred_element_type=_f32)
    xs_ref[...] = xw * dinv[:, None]


def _mm23(acc, dinv3, b, w):
    return pl.pallas_call(
        _mm23_body,
        grid=(NB,),
        in_specs=[
            pl.BlockSpec((RB, H), lambda i: (i, 0)),
            pl.BlockSpec((1, 1, RB), lambda i: (i, 0, 0)),
            pl.BlockSpec((1, H), lambda i: (0, 0)),
            pl.BlockSpec((H, H), lambda i: (0, 0)),
        ],
        out_specs=pl.BlockSpec((RB, H), lambda i: (i, 0)),
        out_shape=jax.ShapeDtypeStruct((N, H), _f32),
    )(acc, dinv3, b, w)


def _pool_body(acc_ref, dinv_ref, b3_ref, batch_ref,
               wf1_ref, bf1_ref, wf2t_ref, bf2_ref, out_ref, sums, cnt):
    i = pl.program_id(0)

    @pl.when(i == 0)
    def _():
        sums[...] = jnp.zeros_like(sums)
        cnt[...] = jnp.zeros_like(cnt)

    dinv = dinv_ref[0, 0, :]
    h = _leaky(acc_ref[...] * dinv[:, None] + b3_ref[0, :])
    seg = batch_ref[0, 0, :]
    onehot = (seg[:, None]
              == lax.broadcasted_iota(jnp.int32, (RB, G), 1)).astype(_f32)
    sums[...] += lax.dot_general(onehot, h, (((0,), (0,)), ((), ())),
                                 preferred_element_type=_f32)
    cnt[...] += jnp.sum(onehot, axis=0, keepdims=True)

    @pl.when(i == NB - 1)
    def _():
        pooled = sums[...] / jnp.maximum(cnt[0, :], 1.0)[:, None]
        hh = _leaky(jnp.dot(pooled, wf1_ref[...],
                            preferred_element_type=_f32) + bf1_ref[0, :])
        o = jnp.sum(hh * wf2t_ref[0, :][None, :], axis=1) + bf2_ref[0, 0]
        out_ref[0, :] = o


def _pool(acc, dinv3, b3, batch3, wf1, bf1, wf2t, bf2):
    return pl.pallas_call(
        _pool_body,
        grid=(NB,),
        in_specs=[
            pl.BlockSpec((RB, H), lambda i: (i, 0)),
            pl.BlockSpec((1, 1, RB), lambda i: (i, 0, 0)),
            pl.BlockSpec((1, H), lambda i: (0, 0)),
            pl.BlockSpec((1, 1, RB), lambda i: (i, 0, 0)),
            pl.BlockSpec((H, H), lambda i: (0, 0)),
            pl.BlockSpec((1, H), lambda i: (0, 0)),
            pl.BlockSpec((1, H), lambda i: (0, 0)),
            pl.BlockSpec((1, 1), lambda i: (0, 0)),
        ],
        out_specs=pl.BlockSpec((1, G), lambda i: (0, 0)),
        out_shape=jax.ShapeDtypeStruct((1, G), _f32),
        scratch_shapes=[
            pltpu.VMEM((G, H), _f32),
            pltpu.VMEM((1, G), _f32),
        ],
    )(acc, dinv3, b3, batch3, wf1, bf1, wf2t, bf2)


def kernel(x, edge_index, batch, W1, b1, W2, b2, W3, b3, Wf1, bf1, Wf2, bf2):
    src1 = edge_index[0]
    dst1 = edge_index[1]
    ones_h = jnp.ones((CH,), _f32)
    zeros_h = jnp.zeros((N,), _f32)

    deg_a, deg_b = _deg_kernel(dst1, ones_h, zeros_h)
    xs, dinv3 = _mm1(x, W1, deg_a.reshape(NB, 1, RB), deg_b.reshape(NB, 1, RB))

    def prop(xs):
        acc_cm = _prop_kernel(xs.T.reshape(H * N), src1, dst1)
        return acc_cm.reshape(H, N).T

    acc = prop(xs)
    xs = _mm23(acc, dinv3, b1.reshape(1, H), W2)
    acc = prop(xs)
    xs = _mm23(acc, dinv3, b2.reshape(1, H), W3)
    acc = prop(xs)

    out = _pool(acc, dinv3, b3.reshape(1, H),
                batch.reshape(NB, 1, RB), Wf1, bf1.reshape(1, H),
                Wf2.reshape(1, H), bf2.reshape(1, 1))
    return out.reshape(G)
